# Initial kernel scaffold; baseline (speedup 1.0000x reference)
import jax
import jax.numpy as jnp
from jax import lax
from jax.experimental import pallas as pl
from jax.experimental.pallas import tpu as pltpu
from jax.experimental.pallas import tpu_sc as plsc

DIM = 32
TOTAL = 16384 * 200
NC, NS = 2, 16
NW = NC * NS
IDX_W = 128
SUB = 8
CHUNK = SUB * IDX_W
ROWS_PER_W = TOTAL // NW
IDXROWS_PER_W = ROWS_PER_W // IDX_W
CHUNKS_PER_W = IDXROWS_PER_W // SUB


def _gather_body(idx_hbm, table_hbm, out_hbm, idx_v, rows_v, sem):
    wid = lax.axis_index("s") * NC + lax.axis_index("c")
    row0 = wid * IDXROWS_PER_W

    def chunk(g, carry):
        r0 = row0 + g * SUB
        pltpu.sync_copy(idx_hbm.at[pl.ds(r0, SUB)], idx_v)
        cps = [
            pltpu.async_copy(
                table_hbm.at[idx_v.at[j]],
                rows_v.at[pl.ds(j * IDX_W, IDX_W)],
                sem,
            )
            for j in range(SUB)
        ]
        for cp in cps:
            cp.wait()
        pltpu.sync_copy(rows_v, out_hbm.at[pl.ds(r0 * IDX_W, CHUNK)])
        return carry

    lax.fori_loop(0, CHUNKS_PER_W, chunk, 0)


def kernel(input, weight):
    idx2d = input.reshape(TOTAL // IDX_W, IDX_W).astype(jnp.int32)
    mesh = plsc.VectorSubcoreMesh(core_axis_name="c", subcore_axis_name="s")
    flat = pl.kernel(
        _gather_body,
        mesh=mesh,
        out_type=jax.ShapeDtypeStruct((TOTAL, DIM), jnp.float32),
        scratch_types=[
            pltpu.VMEM((SUB, IDX_W), jnp.int32),
            pltpu.VMEM((CHUNK, DIM), jnp.float32),
            pltpu.SemaphoreType.DMA,
        ],
    )(idx2d, weight)
    return flat.reshape(input.shape[0], input.shape[1], DIM)


# speedup vs baseline: 4.8036x; 4.8036x over previous
import jax
import jax.numpy as jnp
from jax import lax
from jax.experimental import pallas as pl
from jax.experimental.pallas import tpu as pltpu
from jax.experimental.pallas import tpu_sc as plsc

DIM = 32
TOTAL = 16384 * 200
NC, NS = 2, 16
NW = NC * NS
IDX_W = 128
SUB = 8
CHUNK = SUB * IDX_W
ROWS_PER_W = TOTAL // NW
IDXROWS_PER_W = ROWS_PER_W // IDX_W
CHUNKS_PER_W = IDXROWS_PER_W // SUB


def _gather_body(idx_hbm, table_hbm, out_hbm, idx_v, rows_v, sem):
    wid = lax.axis_index("s") * NC + lax.axis_index("c")
    row0 = wid * IDXROWS_PER_W

    def chunk(g, carry):
        r0 = row0 + g * SUB
        pltpu.sync_copy(idx_hbm.at[pl.ds(r0, SUB)], idx_v)
        cps = [
            pltpu.async_copy(
                table_hbm.at[idx_v.at[j]],
                rows_v.at[pl.ds(j * IDX_W, IDX_W)],
                sem,
            )
            for j in range(SUB)
        ]
        for cp in cps:
            cp.wait()
        pltpu.sync_copy(rows_v, out_hbm.at[pl.ds(r0 * IDX_W, CHUNK)])
        return carry

    lax.fori_loop(0, CHUNKS_PER_W, chunk, 0)


def kernel(input, weight):
    idx2d = input.reshape(TOTAL // IDX_W, IDX_W).astype(jnp.int32)
    mesh = plsc.VectorSubcoreMesh(core_axis_name="c", subcore_axis_name="s")
    flat = pl.kernel(
        _gather_body,
        mesh=mesh,
        out_type=jax.ShapeDtypeStruct((TOTAL, DIM), jnp.float32),
        scratch_types=[
            pltpu.VMEM((SUB, IDX_W), jnp.int32),
            pltpu.VMEM((CHUNK, DIM), jnp.float32),
            pltpu.SemaphoreType.DMA,
        ],
        compiler_params=pltpu.CompilerParams(use_tc_tiling_on_sc=False),
    )(idx2d, weight)
    return flat.reshape(input.shape[0], input.shape[1], DIM)


# speedup vs baseline: 5.0492x; 1.0511x over previous
import jax
import jax.numpy as jnp
from jax import lax
from jax.experimental import pallas as pl
from jax.experimental.pallas import tpu as pltpu
from jax.experimental.pallas import tpu_sc as plsc

DIM = 32
TOTAL = 16384 * 200
NC, NS = 2, 16
NW = NC * NS
IDX_W = 128
SUB = 8
CHUNK = SUB * IDX_W
ROWS_PER_W = TOTAL // NW
IDXROWS_PER_W = ROWS_PER_W // IDX_W
CHUNKS_PER_W = IDXROWS_PER_W // SUB


def _gather_body(idx_hbm, table_hbm, out_hbm,
                 idx_v0, idx_v1, rows_v0, rows_v1,
                 si0, si1, sg0, sg1, so0, so1):
    wid = lax.axis_index("s") * NC + lax.axis_index("c")
    row0 = wid * IDXROWS_PER_W

    idx_v = (idx_v0, idx_v1)
    rows_v = (rows_v0, rows_v1)
    si = (si0, si1)
    sg = (sg0, sg1)
    so = (so0, so1)

    def idx_slice(g):
        return idx_hbm.at[pl.ds(row0 + g * SUB, SUB)]

    def out_slice(g):
        return out_hbm.at[pl.ds((row0 + g * SUB) * IDX_W, CHUNK)]

    def fire_gathers(b):
        for j in range(SUB):
            pltpu.async_copy(
                table_hbm.at[idx_v[b].at[j]],
                rows_v[b].at[pl.ds(j * IDX_W, IDX_W)],
                sg[b],
            )

    def drain_gathers(b):
        for j in range(SUB):
            pltpu.make_async_copy(
                table_hbm.at[idx_v[b].at[j]],
                rows_v[b].at[pl.ds(j * IDX_W, IDX_W)],
                sg[b],
            ).wait()

    pltpu.async_copy(idx_slice(0), idx_v[0], si[0])

    def round_fn(r, carry):
        for b in range(2):
            g = r * 2 + b
            ob = 1 - b
            pltpu.make_async_copy(idx_slice(g), idx_v[b], si[b]).wait()

            @pl.when(r > 0)
            def _():
                pltpu.make_async_copy(rows_v[b], out_slice(g), so[b]).wait()

            fire_gathers(b)

            @pl.when(g >= 1)
            def _():
                drain_gathers(ob)
                pltpu.async_copy(rows_v[ob], out_slice(g - 1), so[ob])

            @pl.when(g + 1 < CHUNKS_PER_W)
            def _():
                pltpu.async_copy(idx_slice(g + 1), idx_v[ob], si[ob])
        return carry

    lax.fori_loop(0, CHUNKS_PER_W // 2, round_fn, 0)

    last = CHUNKS_PER_W - 1
    drain_gathers(1)
    pltpu.async_copy(rows_v[1], out_slice(last), so[1])
    pltpu.make_async_copy(rows_v[0], out_slice(last - 1), so[0]).wait()
    pltpu.make_async_copy(rows_v[1], out_slice(last), so[1]).wait()


def kernel(input, weight):
    idx2d = input.reshape(TOTAL // IDX_W, IDX_W).astype(jnp.int32)
    mesh = plsc.VectorSubcoreMesh(core_axis_name="c", subcore_axis_name="s")
    flat = pl.kernel(
        _gather_body,
        mesh=mesh,
        out_type=jax.ShapeDtypeStruct((TOTAL, DIM), jnp.float32),
        scratch_types=[
            pltpu.VMEM((SUB, IDX_W), jnp.int32),
            pltpu.VMEM((SUB, IDX_W), jnp.int32),
            pltpu.VMEM((CHUNK, DIM), jnp.float32),
            pltpu.VMEM((CHUNK, DIM), jnp.float32),
            pltpu.SemaphoreType.DMA,
            pltpu.SemaphoreType.DMA,
            pltpu.SemaphoreType.DMA,
            pltpu.SemaphoreType.DMA,
            pltpu.SemaphoreType.DMA,
            pltpu.SemaphoreType.DMA,
        ],
        compiler_params=pltpu.CompilerParams(use_tc_tiling_on_sc=False),
    )(idx2d, weight)
    return flat.reshape(input.shape[0], input.shape[1], DIM)
